# baseline (speedup 1.0000x reference)
import math

import jax
import jax.numpy as jnp
import numpy as np
from jax import lax
from jax.experimental import pallas as pl
from jax.experimental.pallas import tpu as pltpu

IMG = 800.0
STRIDES = (8, 16, 32)
SIZES = (100, 50, 25)
PAD = 100
ANCHOR_SIZES = (32., 64., 128., 256., 512.)
RATIOS = (0.5, 1.0, 2.0)
A = 15
PRE_NMS = 500
POST_NMS = 512
RPN_NMS_THR = 0.7
BOX_NMS_THR = 0.5
SCORE_THR = 0.05
DETS = 100
POOL = 7
SR = 2
CH = 256
NUM_CLASSES = 2
CLIP = math.log(1000.0 / 16)


def _grid_anchors():
    scales = np.array(ANCHOR_SIZES, np.float32)
    hr = np.sqrt(np.array(RATIOS, np.float32)); wr = 1.0 / hr
    ws = (wr[:, None] * scales[None]).reshape(-1)
    hs = (hr[:, None] * scales[None]).reshape(-1)
    base = np.stack([-ws, -hs, ws, hs], 1) / 2.0
    out = []
    for stride, s in zip(STRIDES, SIZES):
        sx = np.arange(s, dtype=np.float32) * stride
        yy, xx = np.meshgrid(sx, sx, indexing='ij')
        shifts = np.stack([xx, yy, xx, yy], -1).reshape(-1, 1, 4)
        out.append(jnp.asarray((shifts + base[None]).reshape(-1, 4)))
    return out


def conv2d(x, w, b, pad):
    y = lax.conv_general_dilated(x, w, (1, 1), pad, dimension_numbers=('NCHW', 'OIHW', 'NCHW'))
    return y + b[None, :, None, None]


def decode(deltas, boxes, weights):
    w = boxes[:, 2] - boxes[:, 0]; h = boxes[:, 3] - boxes[:, 1]
    cx = boxes[:, 0] + 0.5 * w; cy = boxes[:, 1] + 0.5 * h
    dx = deltas[:, 0] / weights[0]; dy = deltas[:, 1] / weights[1]
    dw = jnp.minimum(deltas[:, 2] / weights[2], CLIP)
    dh = jnp.minimum(deltas[:, 3] / weights[3], CLIP)
    pcx = dx * w + cx; pcy = dy * h + cy
    pw = jnp.exp(dw) * w; ph = jnp.exp(dh) * h
    return jnp.stack([pcx - 0.5 * pw, pcy - 0.5 * ph, pcx + 0.5 * pw, pcy + 0.5 * ph], 1)


def _nms_kernel(thr):
    def kern(x1r, y1r, x2r, y2r, keepr):
        x1 = x1r[0]; y1 = y1r[0]; x2 = x2r[0]; y2 = y2r[0]
        n = x1.shape[1]
        area = (x2 - x1) * (y2 - y1)
        iota = lax.broadcasted_iota(jnp.int32, (1, n), 1)

        def body(i, keep):
            onehot = iota == i
            x1i = jnp.sum(jnp.where(onehot, x1, 0.0), axis=1, keepdims=True)
            y1i = jnp.sum(jnp.where(onehot, y1, 0.0), axis=1, keepdims=True)
            x2i = jnp.sum(jnp.where(onehot, x2, 0.0), axis=1, keepdims=True)
            y2i = jnp.sum(jnp.where(onehot, y2, 0.0), axis=1, keepdims=True)
            ai = jnp.sum(jnp.where(onehot, area, 0.0), axis=1, keepdims=True)
            ki = jnp.sum(jnp.where(onehot, keep, 0.0), axis=1, keepdims=True)
            xx1 = jnp.maximum(x1i, x1); yy1 = jnp.maximum(y1i, y1)
            xx2 = jnp.minimum(x2i, x2); yy2 = jnp.minimum(y2i, y2)
            iw = jnp.maximum(xx2 - xx1, 0.0); ih = jnp.maximum(yy2 - yy1, 0.0)
            inter = iw * ih
            iou = inter / (ai + area - inter + 1e-9)
            sup = jnp.where(iou > thr, 1.0, 0.0) * jnp.where(iota > i, 1.0, 0.0) * ki
            return keep * (1.0 - sup)

        keep = lax.fori_loop(0, n, body, jnp.ones((1, n), jnp.float32))
        keepr[0] = keep
    return kern


def _nms_pallas(boxes_sorted, thr):
    B, N, _ = boxes_sorted.shape
    coords = [boxes_sorted[:, :, j].reshape(B, 1, N) for j in range(4)]
    spec = pl.BlockSpec((1, 1, N), lambda b: (b, 0, 0))
    keep = pl.pallas_call(
        _nms_kernel(thr),
        grid=(B,),
        in_specs=[spec] * 4,
        out_specs=spec,
        out_shape=jax.ShapeDtypeStruct((B, 1, N), jnp.float32),
        compiler_params=pltpu.CompilerParams(dimension_semantics=("parallel",)),
    )(*coords)
    return keep.reshape(B, N) > 0.5


def nms_keep(boxes, scores, thr):
    N = boxes.shape[0]
    order = jnp.argsort(-scores)
    npad = ((N + 127) // 128) * 128
    bs = boxes[order]
    bs = jnp.pad(bs, ((0, npad - N), (0, 0)))
    keep_sorted = _nms_pallas(bs[None], thr)[0, :N]
    return jnp.zeros((N,), bool).at[order].set(keep_sorted)


def rpn_proposals(objs, dels, anchors):
    sb, ss, sl = [], [], []
    for lvl in range(3):
        s, i = lax.top_k(objs[lvl], PRE_NMS)
        b = decode(dels[lvl][i], anchors[lvl][i], (1., 1., 1., 1.))
        sb.append(jnp.clip(b, 0.0, IMG)); ss.append(s)
        sl.append(jnp.full((PRE_NMS,), float(lvl), jnp.float32))
    boxes = jnp.concatenate(sb); scores = jnp.concatenate(ss); lvl = jnp.concatenate(sl)
    valid = (boxes[:, 2] - boxes[:, 0] >= 1e-3) & (boxes[:, 3] - boxes[:, 1] >= 1e-3)
    scores = jnp.where(valid, scores, -1e9)
    keep = lax.stop_gradient(nms_keep(boxes + (lvl * (IMG + 16.0))[:, None], scores, RPN_NMS_THR))
    _, top = lax.top_k(jnp.where(keep, scores, -1e9), POST_NMS)
    return boxes[top]


def _roi_prep(boxes):
    sizes = jnp.asarray(SIZES, jnp.float32)
    strides = jnp.asarray(STRIDES, jnp.float32)
    x1 = boxes[:, 0]; y1 = boxes[:, 1]; x2 = boxes[:, 2]; y2 = boxes[:, 3]
    area = jnp.maximum(x2 - x1, 0.0) * jnp.maximum(y2 - y1, 0.0)
    k = jnp.floor(4.0 + jnp.log2(jnp.sqrt(area) / 224.0 + 1e-6))
    lvl = jnp.clip(k, 3.0, 5.0).astype(jnp.int32) - 3
    sc = 1.0 / strides[lvl]; hw = sizes[lvl]
    rx1 = x1 * sc; ry1 = y1 * sc
    rw = jnp.maximum(x2 * sc - rx1, 1.0); rh = jnp.maximum(y2 * sc - ry1, 1.0)
    bw = rw / POOL; bh = rh / POOL
    g = jnp.arange(POOL, dtype=jnp.float32)
    sg = (jnp.arange(SR, dtype=jnp.float32) + 0.5) / SR
    ys = ry1[:, None, None] + (g[None, :, None] + sg[None, None, :]) * bh[:, None, None]
    xs = rx1[:, None, None] + (g[None, :, None] + sg[None, None, :]) * bw[:, None, None]
    Y = jnp.broadcast_to(ys[:, :, None, :, None], (boxes.shape[0], POOL, POOL, SR, SR)).reshape(boxes.shape[0], -1)
    X = jnp.broadcast_to(xs[:, None, :, None, :], (boxes.shape[0], POOL, POOL, SR, SR)).reshape(boxes.shape[0], -1)
    hwb = hw[:, None]
    Y = jnp.clip(Y, 0.0, hwb - 1.0); X = jnp.clip(X, 0.0, hwb - 1.0)
    y0 = jnp.floor(Y); x0 = jnp.floor(X)
    ly = Y - y0; lx = X - x0
    y0i = y0.astype(jnp.int32); x0i = x0.astype(jnp.int32)
    hi = (hwb - 1.0).astype(jnp.int32)
    y1i = jnp.minimum(y0i + 1, hi)
    base = (lvl * (PAD * PAD))[:, None]
    idA = base + y0i * PAD + x0i
    idC = base + y1i * PAD + x0i
    ws = jnp.stack([(1 - ly) * (1 - lx), (1 - ly) * lx, ly * (1 - lx), ly * lx], -1)
    return idA, idC, ws


_FLAT_ROWS = 30720


def _roi_kernel(idA_r, idC_r, ws_r, flat_r, out_r):
    rows = []
    for pq in range(POOL * POOL):
        subs = []
        for u in range(SR * SR):
            s = pq * (SR * SR) + u
            ia = idA_r[0, 0, s]
            ic = idC_r[0, 0, s]
            acc = ws_r[0, 0, 4 * s] * flat_r[ia]
            acc = acc + ws_r[0, 0, 4 * s + 1] * flat_r[ia + 1]
            acc = acc + ws_r[0, 0, 4 * s + 2] * flat_r[ic]
            acc = acc + ws_r[0, 0, 4 * s + 3] * flat_r[ic + 1]
            subs.append(acc)
        m = (subs[0] + subs[2]) + (subs[1] + subs[3])
        rows.append(m * 0.25)
    out_r[0] = jnp.concatenate(rows, axis=0)


def _transpose_kernel(in_r, out_r):
    out_r[0] = jnp.transpose(in_r[0])


def _flat_rows_pallas(flat_T):
    B = flat_T.shape[0]
    chunks = _FLAT_ROWS // 1024
    return pl.pallas_call(
        _transpose_kernel,
        grid=(B, chunks),
        in_specs=[pl.BlockSpec((1, CH, 1024), lambda b, i: (b, 0, i))],
        out_specs=pl.BlockSpec((1, 1024, CH), lambda b, i: (b, i, 0)),
        out_shape=jax.ShapeDtypeStruct((B, _FLAT_ROWS, CH), jnp.float32),
        compiler_params=pltpu.CompilerParams(
            dimension_semantics=("parallel", "arbitrary")),
    )(flat_T)


def _roi_align_pallas(flat_rows, boxes):
    N = boxes.shape[0]
    idA, idC, ws = _roi_prep(boxes)
    ns = POOL * POOL * SR * SR
    idA = idA.reshape(N, 1, ns)
    idC = idC.reshape(N, 1, ns)
    ws = ws.reshape(N, 1, ns * 4)
    out = pl.pallas_call(
        _roi_kernel,
        grid=(N,),
        in_specs=[
            pl.BlockSpec((1, 1, ns), lambda i: (i, 0, 0), memory_space=pltpu.SMEM),
            pl.BlockSpec((1, 1, ns), lambda i: (i, 0, 0), memory_space=pltpu.SMEM),
            pl.BlockSpec((1, 1, ns * 4), lambda i: (i, 0, 0), memory_space=pltpu.SMEM),
            pl.BlockSpec((_FLAT_ROWS, 1, CH), lambda i: (0, 0, 0)),
        ],
        out_specs=pl.BlockSpec((1, POOL * POOL, CH), lambda i: (i, 0, 0)),
        out_shape=jax.ShapeDtypeStruct((N, POOL * POOL, CH), jnp.float32),
        compiler_params=pltpu.CompilerParams(
            dimension_semantics=("arbitrary",),
            vmem_limit_bytes=100 * 1024 * 1024,
        ),
    )(idA, idC, ws, flat_rows.reshape(_FLAT_ROWS, 1, CH))
    return out


def postprocess(cls_logits, deltas, boxes):
    scores = jax.nn.softmax(cls_logits, -1)[:, 1]
    pb = jnp.clip(decode(deltas.reshape(-1, NUM_CLASSES, 4)[:, 1], boxes, (10., 10., 5., 5.)), 0.0, IMG)
    valid = (scores > SCORE_THR) & (pb[:, 2] - pb[:, 0] >= 1e-2) & (pb[:, 3] - pb[:, 1] >= 1e-2)
    s = jnp.where(valid, scores, -1e9)
    keep = lax.stop_gradient(nms_keep(pb, s, BOX_NMS_THR)) & valid
    s = jnp.where(keep, scores, 0.0)
    ts, ti = lax.top_k(s, DETS)
    ok = (ts > 0.0).astype(pb.dtype)
    return jnp.concatenate([pb[ti] * ok[:, None], ts[:, None], ok[:, None]], 1)


def kernel(feat0, feat1, feat2, rpn_conv_w, rpn_conv_b, rpn_cls_w, rpn_cls_b,
           rpn_box_w, rpn_box_b, fc1_w, fc1_b, fc2_w, fc2_b, cls_w, cls_b, box_w, box_b):
    feats = (feat0, feat1, feat2)
    anchors = _grid_anchors()
    B = feat0.shape[0]
    objs, dels = [], []
    for f in feats:
        t = jax.nn.relu(conv2d(f, rpn_conv_w, rpn_conv_b, 'SAME'))
        o = conv2d(t, rpn_cls_w, rpn_cls_b, 'VALID')
        d = conv2d(t, rpn_box_w, rpn_box_b, 'VALID')
        H, W = o.shape[2], o.shape[3]
        objs.append(o.transpose(0, 2, 3, 1).reshape(B, -1))
        dels.append(d.reshape(B, A, 4, H, W).transpose(0, 3, 4, 1, 2).reshape(B, -1, 4))
    props = jax.vmap(lambda o0, o1, o2, d0, d1, d2:
                     rpn_proposals((o0, o1, o2), (d0, d1, d2), anchors))(
        objs[0], objs[1], objs[2], dels[0], dels[1], dels[2])
    f0r = feat0.reshape(B, CH, PAD * PAD)
    f1r = jnp.pad(feat1, ((0, 0), (0, 0), (0, 0), (0, PAD - SIZES[1]))).reshape(B, CH, SIZES[1] * PAD)
    f2r = jnp.pad(feat2, ((0, 0), (0, 0), (0, 0), (0, PAD - SIZES[2]))).reshape(B, CH, SIZES[2] * PAD)
    z1 = jnp.zeros((B, CH, PAD * PAD - SIZES[1] * PAD), jnp.float32)
    z2 = jnp.zeros((B, CH, _FLAT_ROWS - 2 * PAD * PAD - SIZES[2] * PAD), jnp.float32)
    flat_T = jnp.concatenate([f0r, f1r, z1, f2r, z2], axis=2)
    flat_rows = _flat_rows_pallas(flat_T)
    roi_list = [_roi_align_pallas(flat_rows[b], props[b]) for b in range(B)]
    roi = jnp.stack(roi_list)
    x = roi.transpose(0, 1, 3, 2).reshape(B * POST_NMS, CH * POOL * POOL)
    x = jax.nn.relu(x @ fc1_w + fc1_b)
    x = jax.nn.relu(x @ fc2_w + fc2_b)
    cls = (x @ cls_w + cls_b).reshape(B, POST_NMS, NUM_CLASSES)
    dlt = (x @ box_w + box_b).reshape(B, POST_NMS, NUM_CLASSES * 4)
    return jax.vmap(postprocess)(cls, dlt, props)


# speedup vs baseline: 1.0050x; 1.0050x over previous
import math

import jax
import jax.numpy as jnp
import numpy as np
from jax import lax
from jax.experimental import pallas as pl
from jax.experimental.pallas import tpu as pltpu

IMG = 800.0
STRIDES = (8, 16, 32)
SIZES = (100, 50, 25)
PAD = 100
ANCHOR_SIZES = (32., 64., 128., 256., 512.)
RATIOS = (0.5, 1.0, 2.0)
A = 15
PRE_NMS = 500
POST_NMS = 512
RPN_NMS_THR = 0.7
BOX_NMS_THR = 0.5
SCORE_THR = 0.05
DETS = 100
POOL = 7
SR = 2
CH = 256
NUM_CLASSES = 2
CLIP = math.log(1000.0 / 16)


def _grid_anchors():
    scales = np.array(ANCHOR_SIZES, np.float32)
    hr = np.sqrt(np.array(RATIOS, np.float32)); wr = 1.0 / hr
    ws = (wr[:, None] * scales[None]).reshape(-1)
    hs = (hr[:, None] * scales[None]).reshape(-1)
    base = np.stack([-ws, -hs, ws, hs], 1) / 2.0
    out = []
    for stride, s in zip(STRIDES, SIZES):
        sx = np.arange(s, dtype=np.float32) * stride
        yy, xx = np.meshgrid(sx, sx, indexing='ij')
        shifts = np.stack([xx, yy, xx, yy], -1).reshape(-1, 1, 4)
        out.append(jnp.asarray((shifts + base[None]).reshape(-1, 4)))
    return out


def conv2d(x, w, b, pad):
    y = lax.conv_general_dilated(x, w, (1, 1), pad, dimension_numbers=('NCHW', 'OIHW', 'NCHW'))
    return y + b[None, :, None, None]


def decode(deltas, boxes, weights):
    w = boxes[:, 2] - boxes[:, 0]; h = boxes[:, 3] - boxes[:, 1]
    cx = boxes[:, 0] + 0.5 * w; cy = boxes[:, 1] + 0.5 * h
    dx = deltas[:, 0] / weights[0]; dy = deltas[:, 1] / weights[1]
    dw = jnp.minimum(deltas[:, 2] / weights[2], CLIP)
    dh = jnp.minimum(deltas[:, 3] / weights[3], CLIP)
    pcx = dx * w + cx; pcy = dy * h + cy
    pw = jnp.exp(dw) * w; ph = jnp.exp(dh) * h
    return jnp.stack([pcx - 0.5 * pw, pcy - 0.5 * ph, pcx + 0.5 * pw, pcy + 0.5 * ph], 1)


def _nms_kernel(thr):
    def kern(x1r, y1r, x2r, y2r, keepr):
        x1 = x1r[0]; y1 = y1r[0]; x2 = x2r[0]; y2 = y2r[0]
        n = x1.shape[1]
        area = (x2 - x1) * (y2 - y1)
        iota = lax.broadcasted_iota(jnp.int32, (1, n), 1)

        def body(i, keep):
            onehot = iota == i
            x1i = jnp.sum(jnp.where(onehot, x1, 0.0), axis=1, keepdims=True)
            y1i = jnp.sum(jnp.where(onehot, y1, 0.0), axis=1, keepdims=True)
            x2i = jnp.sum(jnp.where(onehot, x2, 0.0), axis=1, keepdims=True)
            y2i = jnp.sum(jnp.where(onehot, y2, 0.0), axis=1, keepdims=True)
            ai = jnp.sum(jnp.where(onehot, area, 0.0), axis=1, keepdims=True)
            ki = jnp.sum(jnp.where(onehot, keep, 0.0), axis=1, keepdims=True)
            xx1 = jnp.maximum(x1i, x1); yy1 = jnp.maximum(y1i, y1)
            xx2 = jnp.minimum(x2i, x2); yy2 = jnp.minimum(y2i, y2)
            iw = jnp.maximum(xx2 - xx1, 0.0); ih = jnp.maximum(yy2 - yy1, 0.0)
            inter = iw * ih
            iou = inter / (ai + area - inter + 1e-9)
            sup = jnp.where(iou > thr, 1.0, 0.0) * jnp.where(iota > i, 1.0, 0.0) * ki
            return keep * (1.0 - sup)

        keep = lax.fori_loop(0, n, body, jnp.ones((1, n), jnp.float32))
        keepr[0] = keep
    return kern


def _nms_pallas(boxes_sorted, thr):
    B, N, _ = boxes_sorted.shape
    coords = [boxes_sorted[:, :, j].reshape(B, 1, N) for j in range(4)]
    spec = pl.BlockSpec((1, 1, N), lambda b: (b, 0, 0))
    keep = pl.pallas_call(
        _nms_kernel(thr),
        grid=(B,),
        in_specs=[spec] * 4,
        out_specs=spec,
        out_shape=jax.ShapeDtypeStruct((B, 1, N), jnp.float32),
        compiler_params=pltpu.CompilerParams(dimension_semantics=("parallel",)),
    )(*coords)
    return keep.reshape(B, N) > 0.5


def nms_keep(boxes, scores, thr):
    N = boxes.shape[0]
    order = jnp.argsort(-scores)
    npad = ((N + 127) // 128) * 128
    bs = boxes[order]
    bs = jnp.pad(bs, ((0, npad - N), (0, 0)))
    keep_sorted = _nms_pallas(bs[None], thr)[0, :N]
    return jnp.zeros((N,), bool).at[order].set(keep_sorted)


def rpn_proposals(objs, dels, anchors):
    sb, ss, sl = [], [], []
    for lvl in range(3):
        s, i = lax.top_k(objs[lvl], PRE_NMS)
        b = decode(dels[lvl][i], anchors[lvl][i], (1., 1., 1., 1.))
        sb.append(jnp.clip(b, 0.0, IMG)); ss.append(s)
        sl.append(jnp.full((PRE_NMS,), float(lvl), jnp.float32))
    boxes = jnp.concatenate(sb); scores = jnp.concatenate(ss); lvl = jnp.concatenate(sl)
    valid = (boxes[:, 2] - boxes[:, 0] >= 1e-3) & (boxes[:, 3] - boxes[:, 1] >= 1e-3)
    scores = jnp.where(valid, scores, -1e9)
    keep = lax.stop_gradient(nms_keep(boxes + (lvl * (IMG + 16.0))[:, None], scores, RPN_NMS_THR))
    _, top = lax.top_k(jnp.where(keep, scores, -1e9), POST_NMS)
    return boxes[top]


def _roi_prep(boxes):
    sizes = jnp.asarray(SIZES, jnp.float32)
    strides = jnp.asarray(STRIDES, jnp.float32)
    x1 = boxes[:, 0]; y1 = boxes[:, 1]; x2 = boxes[:, 2]; y2 = boxes[:, 3]
    area = jnp.maximum(x2 - x1, 0.0) * jnp.maximum(y2 - y1, 0.0)
    k = jnp.floor(4.0 + jnp.log2(jnp.sqrt(area) / 224.0 + 1e-6))
    lvl = jnp.clip(k, 3.0, 5.0).astype(jnp.int32) - 3
    sc = 1.0 / strides[lvl]; hw = sizes[lvl]
    rx1 = x1 * sc; ry1 = y1 * sc
    rw = jnp.maximum(x2 * sc - rx1, 1.0); rh = jnp.maximum(y2 * sc - ry1, 1.0)
    bw = rw / POOL; bh = rh / POOL
    g = jnp.arange(POOL, dtype=jnp.float32)
    sg = (jnp.arange(SR, dtype=jnp.float32) + 0.5) / SR
    ys = ry1[:, None, None] + (g[None, :, None] + sg[None, None, :]) * bh[:, None, None]
    xs = rx1[:, None, None] + (g[None, :, None] + sg[None, None, :]) * bw[:, None, None]
    Y = jnp.broadcast_to(ys[:, :, None, :, None], (boxes.shape[0], POOL, POOL, SR, SR)).reshape(boxes.shape[0], -1)
    X = jnp.broadcast_to(xs[:, None, :, None, :], (boxes.shape[0], POOL, POOL, SR, SR)).reshape(boxes.shape[0], -1)
    hwb = hw[:, None]
    Y = jnp.clip(Y, 0.0, hwb - 1.0); X = jnp.clip(X, 0.0, hwb - 1.0)
    y0 = jnp.floor(Y); x0 = jnp.floor(X)
    ly = Y - y0; lx = X - x0
    y0i = y0.astype(jnp.int32); x0i = x0.astype(jnp.int32)
    hi = (hwb - 1.0).astype(jnp.int32)
    y1i = jnp.minimum(y0i + 1, hi)
    base = (lvl * (PAD * PAD))[:, None]
    idA = base + y0i * PAD + x0i
    idC = base + y1i * PAD + x0i
    ws = jnp.stack([(1 - ly) * (1 - lx), (1 - ly) * lx, ly * (1 - lx), ly * lx], -1)
    return idA, idC, ws


_FLAT_ROWS = 30720


def _roi_kernel(idA_r, idC_r, ws_r, flat_r, out_r):
    rows = []
    for pq in range(POOL * POOL):
        subs = []
        for u in range(SR * SR):
            s = pq * (SR * SR) + u
            ia = idA_r[0, 0, s]
            ic = idC_r[0, 0, s]
            acc = ws_r[0, 0, 4 * s] * flat_r[ia]
            acc = acc + ws_r[0, 0, 4 * s + 1] * flat_r[ia + 1]
            acc = acc + ws_r[0, 0, 4 * s + 2] * flat_r[ic]
            acc = acc + ws_r[0, 0, 4 * s + 3] * flat_r[ic + 1]
            subs.append(acc)
        m = (subs[0] + subs[2]) + (subs[1] + subs[3])
        rows.append(m * 0.25)
    out_r[0] = jnp.concatenate(rows, axis=0)


def _transpose_kernel(in_r, out_r):
    out_r[0] = jnp.transpose(in_r[0])[:, None, :]


def _flat_rows_pallas(flat_T):
    B = flat_T.shape[0]
    chunks = _FLAT_ROWS // 1024
    return pl.pallas_call(
        _transpose_kernel,
        grid=(B, chunks),
        in_specs=[pl.BlockSpec((1, CH, 1024), lambda b, i: (b, 0, i))],
        out_specs=pl.BlockSpec((1, 1024, 1, CH), lambda b, i: (b, i, 0, 0)),
        out_shape=jax.ShapeDtypeStruct((B, _FLAT_ROWS, 1, CH), jnp.float32),
        compiler_params=pltpu.CompilerParams(
            dimension_semantics=("parallel", "arbitrary")),
    )(flat_T)


def _roi_align_pallas(flat_rows, boxes):
    N = boxes.shape[0]
    idA, idC, ws = _roi_prep(boxes)
    ns = POOL * POOL * SR * SR
    idA = idA.reshape(N, 1, ns)
    idC = idC.reshape(N, 1, ns)
    ws = ws.reshape(N, 1, ns * 4)
    out = pl.pallas_call(
        _roi_kernel,
        grid=(N,),
        in_specs=[
            pl.BlockSpec((1, 1, ns), lambda i: (i, 0, 0), memory_space=pltpu.SMEM),
            pl.BlockSpec((1, 1, ns), lambda i: (i, 0, 0), memory_space=pltpu.SMEM),
            pl.BlockSpec((1, 1, ns * 4), lambda i: (i, 0, 0), memory_space=pltpu.SMEM),
            pl.BlockSpec((_FLAT_ROWS, 1, CH), lambda i: (0, 0, 0)),
        ],
        out_specs=pl.BlockSpec((1, POOL * POOL, CH), lambda i: (i, 0, 0)),
        out_shape=jax.ShapeDtypeStruct((N, POOL * POOL, CH), jnp.float32),
        compiler_params=pltpu.CompilerParams(
            dimension_semantics=("arbitrary",),
            vmem_limit_bytes=100 * 1024 * 1024,
        ),
    )(idA, idC, ws, flat_rows)
    return out


def postprocess(cls_logits, deltas, boxes):
    scores = jax.nn.softmax(cls_logits, -1)[:, 1]
    pb = jnp.clip(decode(deltas.reshape(-1, NUM_CLASSES, 4)[:, 1], boxes, (10., 10., 5., 5.)), 0.0, IMG)
    valid = (scores > SCORE_THR) & (pb[:, 2] - pb[:, 0] >= 1e-2) & (pb[:, 3] - pb[:, 1] >= 1e-2)
    s = jnp.where(valid, scores, -1e9)
    keep = lax.stop_gradient(nms_keep(pb, s, BOX_NMS_THR)) & valid
    s = jnp.where(keep, scores, 0.0)
    ts, ti = lax.top_k(s, DETS)
    ok = (ts > 0.0).astype(pb.dtype)
    return jnp.concatenate([pb[ti] * ok[:, None], ts[:, None], ok[:, None]], 1)


def kernel(feat0, feat1, feat2, rpn_conv_w, rpn_conv_b, rpn_cls_w, rpn_cls_b,
           rpn_box_w, rpn_box_b, fc1_w, fc1_b, fc2_w, fc2_b, cls_w, cls_b, box_w, box_b):
    feats = (feat0, feat1, feat2)
    anchors = _grid_anchors()
    B = feat0.shape[0]
    objs, dels = [], []
    for f in feats:
        t = jax.nn.relu(conv2d(f, rpn_conv_w, rpn_conv_b, 'SAME'))
        o = conv2d(t, rpn_cls_w, rpn_cls_b, 'VALID')
        d = conv2d(t, rpn_box_w, rpn_box_b, 'VALID')
        H, W = o.shape[2], o.shape[3]
        objs.append(o.transpose(0, 2, 3, 1).reshape(B, -1))
        dels.append(d.reshape(B, A, 4, H, W).transpose(0, 3, 4, 1, 2).reshape(B, -1, 4))
    props = jax.vmap(lambda o0, o1, o2, d0, d1, d2:
                     rpn_proposals((o0, o1, o2), (d0, d1, d2), anchors))(
        objs[0], objs[1], objs[2], dels[0], dels[1], dels[2])
    f0r = feat0.reshape(B, CH, PAD * PAD)
    f1r = jnp.pad(feat1, ((0, 0), (0, 0), (0, 0), (0, PAD - SIZES[1]))).reshape(B, CH, SIZES[1] * PAD)
    f2r = jnp.pad(feat2, ((0, 0), (0, 0), (0, 0), (0, PAD - SIZES[2]))).reshape(B, CH, SIZES[2] * PAD)
    z1 = jnp.zeros((B, CH, PAD * PAD - SIZES[1] * PAD), jnp.float32)
    z2 = jnp.zeros((B, CH, _FLAT_ROWS - 2 * PAD * PAD - SIZES[2] * PAD), jnp.float32)
    flat_T = jnp.concatenate([f0r, f1r, z1, f2r, z2], axis=2)
    flat_rows = _flat_rows_pallas(flat_T)
    roi_list = [_roi_align_pallas(flat_rows[b], props[b]) for b in range(B)]
    roi = jnp.stack(roi_list)
    x = roi.transpose(0, 1, 3, 2).reshape(B * POST_NMS, CH * POOL * POOL)
    x = jax.nn.relu(x @ fc1_w + fc1_b)
    x = jax.nn.relu(x @ fc2_w + fc2_b)
    cls = (x @ cls_w + cls_b).reshape(B, POST_NMS, NUM_CLASSES)
    dlt = (x @ box_w + box_b).reshape(B, POST_NMS, NUM_CLASSES * 4)
    return jax.vmap(postprocess)(cls, dlt, props)


# speedup vs baseline: 3.3698x; 3.3532x over previous
import math

import jax
import jax.numpy as jnp
import numpy as np
from jax import lax
from jax.experimental import pallas as pl
from jax.experimental.pallas import tpu as pltpu

IMG = 800.0
STRIDES = (8, 16, 32)
SIZES = (100, 50, 25)
PAD = 100
ANCHOR_SIZES = (32., 64., 128., 256., 512.)
RATIOS = (0.5, 1.0, 2.0)
A = 15
PRE_NMS = 500
POST_NMS = 512
RPN_NMS_THR = 0.7
BOX_NMS_THR = 0.5
SCORE_THR = 0.05
DETS = 100
POOL = 7
SR = 2
CH = 256
NUM_CLASSES = 2
CLIP = math.log(1000.0 / 16)


def _grid_anchors():
    scales = np.array(ANCHOR_SIZES, np.float32)
    hr = np.sqrt(np.array(RATIOS, np.float32)); wr = 1.0 / hr
    ws = (wr[:, None] * scales[None]).reshape(-1)
    hs = (hr[:, None] * scales[None]).reshape(-1)
    base = np.stack([-ws, -hs, ws, hs], 1) / 2.0
    out = []
    for stride, s in zip(STRIDES, SIZES):
        sx = np.arange(s, dtype=np.float32) * stride
        yy, xx = np.meshgrid(sx, sx, indexing='ij')
        shifts = np.stack([xx, yy, xx, yy], -1).reshape(-1, 1, 4)
        out.append(jnp.asarray((shifts + base[None]).reshape(-1, 4)))
    return out


def conv2d(x, w, b, pad):
    y = lax.conv_general_dilated(x, w, (1, 1), pad, dimension_numbers=('NCHW', 'OIHW', 'NCHW'))
    return y + b[None, :, None, None]


def decode(deltas, boxes, weights):
    w = boxes[:, 2] - boxes[:, 0]; h = boxes[:, 3] - boxes[:, 1]
    cx = boxes[:, 0] + 0.5 * w; cy = boxes[:, 1] + 0.5 * h
    dx = deltas[:, 0] / weights[0]; dy = deltas[:, 1] / weights[1]
    dw = jnp.minimum(deltas[:, 2] / weights[2], CLIP)
    dh = jnp.minimum(deltas[:, 3] / weights[3], CLIP)
    pcx = dx * w + cx; pcy = dy * h + cy
    pw = jnp.exp(dw) * w; ph = jnp.exp(dh) * h
    return jnp.stack([pcx - 0.5 * pw, pcy - 0.5 * ph, pcx + 0.5 * pw, pcy + 0.5 * ph], 1)


def _nms_kernel(thr):
    def kern(x1r, y1r, x2r, y2r, keepr):
        x1 = x1r[0]; y1 = y1r[0]; x2 = x2r[0]; y2 = y2r[0]
        n = x1.shape[1]
        area = (x2 - x1) * (y2 - y1)
        iota = lax.broadcasted_iota(jnp.int32, (1, n), 1)

        def body(i, keep):
            onehot = iota == i
            x1i = jnp.sum(jnp.where(onehot, x1, 0.0), axis=1, keepdims=True)
            y1i = jnp.sum(jnp.where(onehot, y1, 0.0), axis=1, keepdims=True)
            x2i = jnp.sum(jnp.where(onehot, x2, 0.0), axis=1, keepdims=True)
            y2i = jnp.sum(jnp.where(onehot, y2, 0.0), axis=1, keepdims=True)
            ai = jnp.sum(jnp.where(onehot, area, 0.0), axis=1, keepdims=True)
            ki = jnp.sum(jnp.where(onehot, keep, 0.0), axis=1, keepdims=True)
            xx1 = jnp.maximum(x1i, x1); yy1 = jnp.maximum(y1i, y1)
            xx2 = jnp.minimum(x2i, x2); yy2 = jnp.minimum(y2i, y2)
            iw = jnp.maximum(xx2 - xx1, 0.0); ih = jnp.maximum(yy2 - yy1, 0.0)
            inter = iw * ih
            iou = inter / (ai + area - inter + 1e-9)
            sup = jnp.where(iou > thr, 1.0, 0.0) * jnp.where(iota > i, 1.0, 0.0) * ki
            return keep * (1.0 - sup)

        keep = lax.fori_loop(0, n, body, jnp.ones((1, n), jnp.float32))
        keepr[0] = keep
    return kern


def _nms_pallas(boxes_sorted, thr):
    B, N, _ = boxes_sorted.shape
    coords = [boxes_sorted[:, :, j].reshape(B, 1, N) for j in range(4)]
    spec = pl.BlockSpec((1, 1, N), lambda b: (b, 0, 0))
    keep = pl.pallas_call(
        _nms_kernel(thr),
        grid=(B,),
        in_specs=[spec] * 4,
        out_specs=spec,
        out_shape=jax.ShapeDtypeStruct((B, 1, N), jnp.float32),
        compiler_params=pltpu.CompilerParams(dimension_semantics=("parallel",)),
    )(*coords)
    return keep.reshape(B, N) > 0.5


def nms_keep(boxes, scores, thr):
    N = boxes.shape[0]
    order = jnp.argsort(-scores)
    npad = ((N + 127) // 128) * 128
    bs = boxes[order]
    bs = jnp.pad(bs, ((0, npad - N), (0, 0)))
    keep_sorted = _nms_pallas(bs[None], thr)[0, :N]
    return jnp.zeros((N,), bool).at[order].set(keep_sorted)


def rpn_proposals(objs, dels, anchors):
    sb, ss, sl = [], [], []
    for lvl in range(3):
        s, i = lax.top_k(objs[lvl], PRE_NMS)
        b = decode(dels[lvl][i], anchors[lvl][i], (1., 1., 1., 1.))
        sb.append(jnp.clip(b, 0.0, IMG)); ss.append(s)
        sl.append(jnp.full((PRE_NMS,), float(lvl), jnp.float32))
    boxes = jnp.concatenate(sb); scores = jnp.concatenate(ss); lvl = jnp.concatenate(sl)
    valid = (boxes[:, 2] - boxes[:, 0] >= 1e-3) & (boxes[:, 3] - boxes[:, 1] >= 1e-3)
    scores = jnp.where(valid, scores, -1e9)
    keep = lax.stop_gradient(nms_keep(boxes + (lvl * (IMG + 16.0))[:, None], scores, RPN_NMS_THR))
    _, top = lax.top_k(jnp.where(keep, scores, -1e9), POST_NMS)
    return boxes[top]


def _roi_prep(boxes):
    sizes = jnp.asarray(SIZES, jnp.float32)
    strides = jnp.asarray(STRIDES, jnp.float32)
    x1 = boxes[:, 0]; y1 = boxes[:, 1]; x2 = boxes[:, 2]; y2 = boxes[:, 3]
    area = jnp.maximum(x2 - x1, 0.0) * jnp.maximum(y2 - y1, 0.0)
    k = jnp.floor(4.0 + jnp.log2(jnp.sqrt(area) / 224.0 + 1e-6))
    lvl = jnp.clip(k, 3.0, 5.0).astype(jnp.int32) - 3
    sc = 1.0 / strides[lvl]; hw = sizes[lvl]
    rx1 = x1 * sc; ry1 = y1 * sc
    rw = jnp.maximum(x2 * sc - rx1, 1.0); rh = jnp.maximum(y2 * sc - ry1, 1.0)
    bw = rw / POOL; bh = rh / POOL
    g = jnp.arange(POOL, dtype=jnp.float32)
    sg = (jnp.arange(SR, dtype=jnp.float32) + 0.5) / SR
    ys = ry1[:, None, None] + (g[None, :, None] + sg[None, None, :]) * bh[:, None, None]
    xs = rx1[:, None, None] + (g[None, :, None] + sg[None, None, :]) * bw[:, None, None]
    Y = jnp.broadcast_to(ys[:, :, None, :, None], (boxes.shape[0], POOL, POOL, SR, SR)).reshape(boxes.shape[0], -1)
    X = jnp.broadcast_to(xs[:, None, :, None, :], (boxes.shape[0], POOL, POOL, SR, SR)).reshape(boxes.shape[0], -1)
    hwb = hw[:, None]
    Y = jnp.clip(Y, 0.0, hwb - 1.0); X = jnp.clip(X, 0.0, hwb - 1.0)
    y0 = jnp.floor(Y); x0 = jnp.floor(X)
    ly = Y - y0; lx = X - x0
    y0i = y0.astype(jnp.int32); x0i = x0.astype(jnp.int32)
    hi = (hwb - 1.0).astype(jnp.int32)
    y1i = jnp.minimum(y0i + 1, hi)
    base = jnp.asarray([0, _L1_BASE, _L2_BASE], jnp.int32)[lvl][:, None]
    idA = base + y0i * PAD + x0i
    idC = base + y1i * PAD + x0i
    ws = jnp.stack([(1 - ly) * (1 - lx), (1 - ly) * lx, ly * (1 - lx), ly * lx], -1)
    return idA, idC, ws


_L1_BASE = 10000
_L2_BASE = 15024
_FLAT_ROWS = 18432


def _roi_kernel(idA_r, idC_r, ws_r, flat_r, out_r):
    rows = []
    for pq in range(POOL * POOL):
        subs = []
        for u in range(SR * SR):
            s = pq * (SR * SR) + u
            ia = idA_r[0, 0, s]
            ic = idC_r[0, 0, s]
            acc = ws_r[0, 0, 4 * s] * flat_r[ia]
            acc = acc + ws_r[0, 0, 4 * s + 1] * flat_r[ia + 1]
            acc = acc + ws_r[0, 0, 4 * s + 2] * flat_r[ic]
            acc = acc + ws_r[0, 0, 4 * s + 3] * flat_r[ic + 1]
            subs.append(acc)
        m = (subs[0] + subs[2]) + (subs[1] + subs[3])
        rows.append(m * 0.25)
    out_r[0] = jnp.concatenate(rows, axis=0)


def _transpose_kernel(in_r, out_r):
    out_r[0] = jnp.transpose(in_r[0])[:, None, :]


def _flat_rows_pallas(flat_T):
    B = flat_T.shape[0]
    chunks = _FLAT_ROWS // 1024
    return pl.pallas_call(
        _transpose_kernel,
        grid=(B, chunks),
        in_specs=[pl.BlockSpec((1, CH, 1024), lambda b, i: (b, 0, i))],
        out_specs=pl.BlockSpec((1, 1024, 1, CH), lambda b, i: (b, i, 0, 0)),
        out_shape=jax.ShapeDtypeStruct((B, _FLAT_ROWS, 1, CH), jnp.float32),
        compiler_params=pltpu.CompilerParams(
            dimension_semantics=("parallel", "arbitrary")),
    )(flat_T)


def _roi_align_pallas(flat_rows, props):
    B, N = props.shape[0], props.shape[1]
    idA_l, idC_l, ws_l = [], [], []
    for b in range(B):
        idA, idC, ws = _roi_prep(props[b])
        idA_l.append(idA + b * _FLAT_ROWS)
        idC_l.append(idC + b * _FLAT_ROWS)
        ws_l.append(ws)
    idA = jnp.concatenate(idA_l); idC = jnp.concatenate(idC_l)
    ws = jnp.concatenate(ws_l)
    M = B * N
    ns = POOL * POOL * SR * SR
    idA = idA.reshape(M, 1, ns)
    idC = idC.reshape(M, 1, ns)
    ws = ws.reshape(M, 1, ns * 4)
    out = pl.pallas_call(
        _roi_kernel,
        grid=(M,),
        in_specs=[
            pl.BlockSpec((1, 1, ns), lambda i: (i, 0, 0), memory_space=pltpu.SMEM),
            pl.BlockSpec((1, 1, ns), lambda i: (i, 0, 0), memory_space=pltpu.SMEM),
            pl.BlockSpec((1, 1, ns * 4), lambda i: (i, 0, 0), memory_space=pltpu.SMEM),
            pl.BlockSpec((B * _FLAT_ROWS, 1, CH), lambda i: (0, 0, 0)),
        ],
        out_specs=pl.BlockSpec((1, POOL * POOL, CH), lambda i: (i, 0, 0)),
        out_shape=jax.ShapeDtypeStruct((M, POOL * POOL, CH), jnp.float32),
        compiler_params=pltpu.CompilerParams(
            dimension_semantics=("arbitrary",),
            vmem_limit_bytes=100 * 1024 * 1024,
        ),
    )(idA, idC, ws, flat_rows)
    return out


def postprocess(cls_logits, deltas, boxes):
    scores = jax.nn.softmax(cls_logits, -1)[:, 1]
    pb = jnp.clip(decode(deltas.reshape(-1, NUM_CLASSES, 4)[:, 1], boxes, (10., 10., 5., 5.)), 0.0, IMG)
    valid = (scores > SCORE_THR) & (pb[:, 2] - pb[:, 0] >= 1e-2) & (pb[:, 3] - pb[:, 1] >= 1e-2)
    s = jnp.where(valid, scores, -1e9)
    keep = lax.stop_gradient(nms_keep(pb, s, BOX_NMS_THR)) & valid
    s = jnp.where(keep, scores, 0.0)
    ts, ti = lax.top_k(s, DETS)
    ok = (ts > 0.0).astype(pb.dtype)
    return jnp.concatenate([pb[ti] * ok[:, None], ts[:, None], ok[:, None]], 1)


def kernel(feat0, feat1, feat2, rpn_conv_w, rpn_conv_b, rpn_cls_w, rpn_cls_b,
           rpn_box_w, rpn_box_b, fc1_w, fc1_b, fc2_w, fc2_b, cls_w, cls_b, box_w, box_b):
    feats = (feat0, feat1, feat2)
    anchors = _grid_anchors()
    B = feat0.shape[0]
    objs, dels = [], []
    for f in feats:
        t = jax.nn.relu(conv2d(f, rpn_conv_w, rpn_conv_b, 'SAME'))
        o = conv2d(t, rpn_cls_w, rpn_cls_b, 'VALID')
        d = conv2d(t, rpn_box_w, rpn_box_b, 'VALID')
        H, W = o.shape[2], o.shape[3]
        objs.append(o.transpose(0, 2, 3, 1).reshape(B, -1))
        dels.append(d.reshape(B, A, 4, H, W).transpose(0, 3, 4, 1, 2).reshape(B, -1, 4))
    props = jax.vmap(lambda o0, o1, o2, d0, d1, d2:
                     rpn_proposals((o0, o1, o2), (d0, d1, d2), anchors))(
        objs[0], objs[1], objs[2], dels[0], dels[1], dels[2])
    f0r = feat0.reshape(B, CH, PAD * PAD)
    f1r = jnp.pad(feat1, ((0, 0), (0, 0), (0, 0), (0, PAD - SIZES[1]))).reshape(B, CH, SIZES[1] * PAD)
    f2r = jnp.pad(feat2, ((0, 0), (0, 0), (0, 0), (0, PAD - SIZES[2]))).reshape(B, CH, SIZES[2] * PAD)
    z1 = jnp.zeros((B, CH, _L2_BASE - _L1_BASE - SIZES[1] * PAD), jnp.float32)
    z2 = jnp.zeros((B, CH, _FLAT_ROWS - _L2_BASE - SIZES[2] * PAD), jnp.float32)
    flat_T = jnp.concatenate([f0r, f1r, z1, f2r, z2], axis=2)
    flat_rows = _flat_rows_pallas(flat_T)
    roi = _roi_align_pallas(flat_rows.reshape(B * _FLAT_ROWS, 1, CH), props)
    roi = roi.reshape(B, POST_NMS, POOL * POOL, CH)
    x = roi.transpose(0, 1, 3, 2).reshape(B * POST_NMS, CH * POOL * POOL)
    x = jax.nn.relu(x @ fc1_w + fc1_b)
    x = jax.nn.relu(x @ fc2_w + fc2_b)
    cls = (x @ cls_w + cls_b).reshape(B, POST_NMS, NUM_CLASSES)
    dlt = (x @ box_w + box_b).reshape(B, POST_NMS, NUM_CLASSES * 4)
    return jax.vmap(postprocess)(cls, dlt, props)
